# baseline (speedup 1.0000x reference)
import functools

import jax
import jax.numpy as jnp
from jax import lax
from jax.experimental import pallas as pl
from jax.experimental.pallas import tpu as pltpu
from jax.experimental.pallas import tpu_sc as plsc

N_NODES = 1000000
EMB_DIM = 64
BATCH = 16384

_NC = 2
_NS = 16
_NW = _NC * _NS
_BPW = BATCH // _NW
_L = 16


def _body(s_hbm, t_hbm, src_hbm, dst_hbm, rel_hbm, bu_hbm, bv_hbm, out_hbm,
          idx_s, idx_t, u_rows, v_rows, rel_v, bs_v, bt_v, out_v,
          sem_u, sem_v, sem_bs, sem_bt):
    wid = lax.axis_index("s") * _NC + lax.axis_index("c")
    base = wid * _BPW

    pltpu.sync_copy(s_hbm.at[pl.ds(base, _BPW)], idx_s)
    pltpu.sync_copy(t_hbm.at[pl.ds(base, _BPW)], idx_t)
    pltpu.sync_copy(rel_hbm, rel_v)

    cu = pltpu.async_copy(src_hbm.at[idx_s], u_rows, sem_u)
    cv = pltpu.async_copy(dst_hbm.at[idx_t], v_rows, sem_v)
    cbs = pltpu.async_copy(bu_hbm.at[idx_s], bs_v, sem_bs)
    cbt = pltpu.async_copy(bv_hbm.at[idx_t], bt_v, sem_bt)
    cu.wait()
    cv.wait()
    cbs.wait()
    cbt.wait()

    r0 = rel_v[pl.ds(0, _L)]
    r1 = rel_v[pl.ds(_L, _L)]
    r2 = rel_v[pl.ds(2 * _L, _L)]
    r3 = rel_v[pl.ds(3 * _L, _L)]

    mask15 = lax.iota(jnp.int32, _L) == (_L - 1)

    def row(i, carry):
        acc = u_rows[i, pl.ds(0, _L)] * r0 * v_rows[i, pl.ds(0, _L)]
        acc += u_rows[i, pl.ds(_L, _L)] * r1 * v_rows[i, pl.ds(_L, _L)]
        acc += u_rows[i, pl.ds(2 * _L, _L)] * r2 * v_rows[i, pl.ds(2 * _L, _L)]
        acc += u_rows[i, pl.ds(3 * _L, _L)] * r3 * v_rows[i, pl.ds(3 * _L, _L)]
        c = plsc.cumsum(acc)
        plsc.store_scatter(out_v, [jnp.full((_L,), i, jnp.int32)], c, mask=mask15)
        return carry

    lax.fori_loop(0, _BPW, row, 0)

    for blk in range(_BPW // _L):
        sl = pl.ds(blk * _L, _L)
        out_v[sl] = out_v[sl] + bs_v[sl] + bt_v[sl]

    pltpu.sync_copy(out_v, out_hbm.at[pl.ds(base, _BPW)])


@jax.jit
def _run(s, t, src, dst, rel, bu1, bv1):
    mesh = plsc.VectorSubcoreMesh(core_axis_name="c", subcore_axis_name="s")
    kern = functools.partial(
        pl.kernel,
        mesh=mesh,
        compiler_params=pltpu.CompilerParams(
            needs_layout_passes=False, use_tc_tiling_on_sc=False
        ),
        out_type=jax.ShapeDtypeStruct((BATCH,), jnp.float32),
        scratch_types=[
            pltpu.VMEM((_BPW,), jnp.int32),
            pltpu.VMEM((_BPW,), jnp.int32),
            pltpu.VMEM((_BPW, EMB_DIM), jnp.float32),
            pltpu.VMEM((_BPW, EMB_DIM), jnp.float32),
            pltpu.VMEM((EMB_DIM,), jnp.float32),
            pltpu.VMEM((_BPW,), jnp.float32),
            pltpu.VMEM((_BPW,), jnp.float32),
            pltpu.VMEM((_BPW,), jnp.float32),
            pltpu.SemaphoreType.DMA,
            pltpu.SemaphoreType.DMA,
            pltpu.SemaphoreType.DMA,
            pltpu.SemaphoreType.DMA,
        ],
    )(_body)
    return kern(s, t, src, dst, rel, bu1, bv1)


def kernel(pairs, src, dst, rel, bu, bv):
    s = pairs[:, 0].astype(jnp.int32)
    t = pairs[:, 1].astype(jnp.int32)
    bu1 = bu.reshape(N_NODES)
    bv1 = bv.reshape(N_NODES)
    return _run(s, t, src, dst, rel, bu1, bv1)
